# baseline (speedup 1.0000x reference)
import functools

import jax
import jax.numpy as jnp
from jax import lax
from jax.experimental import pallas as pl
from jax.experimental.pallas import tpu as pltpu
from jax.experimental.pallas import tpu_sc as plsc

_NSUB = 16
_NWORKERS = 32
_CE = 10240


@functools.lru_cache(maxsize=None)
def _edge_pass_kernel(n, d, e_pad, ncols):
    col_groups = d // ncols
    wpc = _NWORKERS // col_groups
    ew = e_pad // wpc
    nchunks = ew // _CE
    n_pad = n + _NSUB
    groups = _CE // 16
    unroll = 8
    mesh = plsc.VectorSubcoreMesh(core_axis_name="c", subcore_axis_name="s")

    @functools.partial(
        pl.kernel,
        out_type=jax.ShapeDtypeStruct((wpc * d * n,), jnp.float32),
        mesh=mesh,
        compiler_params=pltpu.CompilerParams(needs_layout_passes=False),
        scratch_types=(
            [pltpu.VMEM((n_pad,), jnp.float32)] * ncols
            + [pltpu.VMEM((n_pad,), jnp.float32)] * ncols
            + [pltpu.VMEM((_CE,), jnp.int32)] * 4
            + [pltpu.SemaphoreType.DMA] * 4
        ),
    )
    def kern(tflat_hbm, zeros_hbm, src_hbm, dst_hbm, out_hbm, *scr):
        tvs = scr[:ncols]
        avs = scr[ncols:2 * ncols]
        sv0, dv0, sv1, dv1 = scr[2 * ncols:2 * ncols + 4]
        sems = scr[2 * ncols + 4:]
        c = lax.axis_index("c")
        s = lax.axis_index("s")
        wid = c * _NSUB + s
        cg = wid % col_groups
        shard = wid // col_groups
        ebase = shard * ew

        for i in range(ncols):
            coff = pl.multiple_of((cg * ncols + i) * n, 8)
            pltpu.sync_copy(tflat_hbm.at[pl.ds(coff, n)],
                            tvs[i].at[pl.ds(0, n)])
            pltpu.sync_copy(zeros_hbm, avs[i].at[pl.ds(0, n)])

        def start(k, sv, dv, sem_s, sem_d):
            off = pl.multiple_of(ebase + k * _CE, 8)
            pltpu.async_copy(src_hbm.at[pl.ds(off, _CE)], sv, sem_s)
            pltpu.async_copy(dst_hbm.at[pl.ds(off, _CE)], dv, sem_d)

        def wait(sv, dv, sem_s, sem_d):
            pltpu.make_async_copy(src_hbm.at[pl.ds(0, _CE)], sv, sem_s).wait()
            pltpu.make_async_copy(dst_hbm.at[pl.ds(0, _CE)], dv, sem_d).wait()

        def process(sv, dv, carry):
            def gbody(g8, carry2):
                for u in range(unroll):
                    goff = pl.multiple_of((g8 * unroll + u) * 16, 16)
                    si = sv[pl.ds(goff, 16)]
                    di = dv[pl.ds(goff, 16)]
                    for i in range(ncols):
                        vals = plsc.load_gather(tvs[i], [si])
                        plsc.addupdate_scatter(avs[i], [di], vals)
                return carry2
            return lax.fori_loop(0, groups // unroll, gbody, carry)

        if nchunks == 1:
            start(0, sv0, dv0, sems[0], sems[1])
            wait(sv0, dv0, sems[0], sems[1])
            process(sv0, dv0, 0)
        else:
            start(0, sv0, dv0, sems[0], sems[1])

            def chunk2_body(k2, carry):
                wait(sv0, dv0, sems[0], sems[1])
                start(2 * k2 + 1, sv1, dv1, sems[2], sems[3])
                carry = process(sv0, dv0, carry)
                wait(sv1, dv1, sems[2], sems[3])

                @pl.when(2 * k2 + 2 < nchunks)
                def _start_next():
                    start(2 * k2 + 2, sv0, dv0, sems[0], sems[1])

                return process(sv1, dv1, carry)

            lax.fori_loop(0, nchunks // 2, chunk2_body, 0)

        for i in range(ncols):
            ooff = pl.multiple_of((shard * d + cg * ncols + i) * n, 8)
            pltpu.sync_copy(avs[i].at[pl.ds(0, n)],
                            out_hbm.at[pl.ds(ooff, n)])

    return kern


def _edge_pass(n, d, e_pad, ncols=1):
    kern = _edge_pass_kernel(n, d, e_pad, ncols)
    wpc = _NWORKERS // (d // ncols)

    def run(table_t_flat, zeros, src_flat, dst_flat):
        out = kern(table_t_flat, zeros, src_flat, dst_flat)
        return out.reshape(wpc, d, n)

    return run


def _leaky(v):
    return jnp.where(v >= 0, v, 0.01 * v)


def _dgT(a, b):
    return lax.dot_general(a, b, (((0,), (0,)), ((), ())),
                           preferred_element_type=jnp.float32)


def _mm_k(x_ref, w_ref, o_ref):
    o_ref[...] = lax.dot_general(w_ref[...], x_ref[...],
                                 (((0,), (1,)), ((), ())),
                                 preferred_element_type=jnp.float32)


def _bn1_k(aggp_ref, b_ref, g_ref, be_ref, rb_ref, ow_ref,
           h_ref, ho_ref):
    v = jnp.sum(aggp_ref[...], axis=0) + b_ref[...][:, None]
    m = jnp.mean(v, axis=1, keepdims=True)
    var = jnp.mean(v * v, axis=1, keepdims=True) - m * m
    h = _leaky((v - m) * (g_ref[...][:, None] * lax.rsqrt(var + 1e-5))
               + be_ref[...][:, None])
    h_ref[...] = h
    ho_ref[...] = _dgT(ow_ref[...], h) + rb_ref[...][:, None]


def _mono_key(score):
    u = lax.bitcast_convert_type(score, jnp.uint32)
    return u ^ jnp.where(u >> 31 != 0,
                         jnp.uint32(0xFFFFFFFF), jnp.uint32(0x80000000))


def _topk_mask(score, valid, k):
    n = score.shape[1]
    key = _mono_key(score)

    def vbit(b, t):
        tt = t | (jnp.uint32(1) << b)
        cnt = jnp.sum(valid & (key >= tt))
        return jnp.where(cnt >= k, tt, t)

    t = lax.fori_loop(0, 32, lambda i, t: vbit(jnp.uint32(31 - i), t),
                      jnp.uint32(0))
    cnt_gt = jnp.sum(valid & (key > t))
    need = k - cnt_gt
    idx = lax.broadcasted_iota(jnp.int32, (1, n), 1)
    ties = valid & (key == t)

    def ibit(b, m):
        mm = m | (jnp.int32(1) << b)
        cnt = jnp.sum(ties & (idx < mm))
        return jnp.where(cnt < need, mm, m)

    m = lax.fori_loop(0, 15, lambda i, m: ibit(jnp.int32(14 - i), m),
                      jnp.int32(0))
    return valid & ((key > t) | (ties & (idx <= m)))


def _topk1_k(saggp_ref, rw_ref, ho_ref, h_ref, w2_ref, h2lin_ref, sel_ref,
             *, k):
    score = _dgT(rw_ref[...], jnp.sum(saggp_ref[...], axis=0)) + ho_ref[...]
    sel = _topk_mask(score, jnp.full(score.shape, True), k)
    xn = jnp.where(sel, h_ref[...] * jnp.tanh(score), 0.0)
    h2lin_ref[...] = _dgT(w2_ref[...], xn)
    sel_ref[...] = sel.astype(jnp.float32)


def _bn2_k(aggp_ref, sel_ref, b_ref, g_ref, be_ref, rb_ref, ow_ref,
           h2_ref, ho_ref, *, k):
    selv = sel_ref[...]
    v = jnp.sum(aggp_ref[...], axis=0) + b_ref[...][:, None]
    m = jnp.sum(v * selv, axis=1, keepdims=True) / k
    var = jnp.sum(v * v * selv, axis=1, keepdims=True) / k - m * m
    h2 = selv * _leaky((v - m) * (g_ref[...][:, None] * lax.rsqrt(var + 1e-5))
                       + be_ref[...][:, None])
    h2_ref[...] = h2
    ho_ref[...] = _dgT(ow_ref[...], h2) + rb_ref[...][:, None]


def _final_k(saggp_ref, rw_ref, ho_ref, sel_ref, h2_ref,
             f1w_ref, f1b_ref, f2w_ref, f2b_ref, f3w_ref, f3b_ref,
             o_ref, *, k):
    score = _dgT(rw_ref[...], jnp.sum(saggp_ref[...], axis=0)) + ho_ref[...]
    sel2 = _topk_mask(score, sel_ref[...] > 0, k)
    w = jnp.where(sel2, jnp.tanh(score), 0.0)
    ap = jnp.sum(h2_ref[...] * w, axis=1, keepdims=True)
    o = _leaky(_dgT(f1w_ref[...], ap) + f1b_ref[...][:, None])
    o = _leaky(_dgT(f2w_ref[...], o) + f2b_ref[...][:, None])
    o = _leaky(_dgT(f3w_ref[...], o) + f3b_ref[...][:, None])
    o_ref[...] = o


def _call(body, out_shapes):
    return pl.pallas_call(
        body, out_shape=[jax.ShapeDtypeStruct(s, jnp.float32)
                         for s in out_shapes])


def kernel(x, edge_index, edge_weigth, batch,
           W1, b1, g1, be1, p1rw, p1rb, p1ow,
           W2, b2, g2, be2, p2rw, p2rb, p2ow,
           f1w, f1b, f2w, f2b, f3w, f3b):
    n, _ = x.shape
    e = edge_index.shape[1]
    k1 = -(-n // 2)
    k2 = -(-k1 // 2)
    d1 = W1.shape[1]
    d2 = W2.shape[1]

    e_pad = -(-e // (_NWORKERS * _CE)) * (_NWORKERS * _CE)
    pad_idx = n + (jnp.arange(e_pad - e, dtype=jnp.int32) % _NSUB)
    src = jnp.concatenate([edge_index[0], pad_idx])
    dst = jnp.concatenate([edge_index[1], pad_idx])

    ep16 = _edge_pass(n, d1, e_pad)
    ep32 = _edge_pass(n, d2, e_pad, ncols=2)
    zeros = jnp.zeros((n,), jnp.float32)

    (hlinT,) = _call(_mm_k, [(d1, n)])(x, W1)
    agg1p = ep16(hlinT.reshape(-1), zeros, src, dst)
    hT, ho1 = _call(_bn1_k, [(d1, n), (1, n)])(
        agg1p, b1, g1, be1, p1rb, p1ow)

    sagg1p = ep16(hT.reshape(-1), zeros, src, dst)
    h2linT, sel1 = _call(functools.partial(_topk1_k, k=k1),
                         [(d2, n), (1, n)])(sagg1p, p1rw, ho1, hT, W2)

    agg2p = ep32(h2linT.reshape(-1), zeros, src, dst)
    h2T, ho2 = _call(functools.partial(_bn2_k, k=k1),
                     [(d2, n), (1, n)])(
        agg2p, sel1, b2, g2, be2, p2rb, p2ow)

    sagg2p = ep32(h2T.reshape(-1), zeros, src, dst)
    (o,) = _call(functools.partial(_final_k, k=k2), [(2, 1)])(
        sagg2p, p2rw, ho2, sel1, h2T, f1w, f1b, f2w, f2b, f3w, f3b)
    return o.T


# speedup vs baseline: 1.9907x; 1.9907x over previous
import functools

import jax
import jax.numpy as jnp
from jax import lax
from jax.experimental import pallas as pl
from jax.experimental.pallas import tpu as pltpu
from jax.experimental.pallas import tpu_sc as plsc

_NSUB = 16
_NWORKERS = 32
_CE = 10240


@functools.lru_cache(maxsize=None)
def _edge_pass_kernel(n, d, e_pad, ncols):
    col_groups = d // ncols
    wpc = _NWORKERS // col_groups
    ew = e_pad // wpc
    nchunks = ew // _CE
    n_pad = n + _NSUB
    groups = _CE // 16
    mesh = plsc.VectorSubcoreMesh(core_axis_name="c", subcore_axis_name="s")

    @functools.partial(
        pl.kernel,
        out_type=jax.ShapeDtypeStruct((wpc * d * n,), jnp.float32),
        mesh=mesh,
        compiler_params=pltpu.CompilerParams(needs_layout_passes=False,
                                             disable_bounds_checks=True),
        scratch_types=(
            [pltpu.VMEM((n_pad,), jnp.float32)] * ncols
            + [pltpu.VMEM((n_pad,), jnp.float32)] * ncols
            + [pltpu.VMEM((_CE,), jnp.int32)] * 4
            + [pltpu.SemaphoreType.DMA] * 4
        ),
    )
    def kern(tflat_hbm, zeros_hbm, src_hbm, dst_hbm, out_hbm, *scr):
        tvs = scr[:ncols]
        avs = scr[ncols:2 * ncols]
        sv0, dv0, sv1, dv1 = scr[2 * ncols:2 * ncols + 4]
        sems = scr[2 * ncols + 4:]
        c = lax.axis_index("c")
        s = lax.axis_index("s")
        wid = c * _NSUB + s
        cg = wid % col_groups
        shard = wid // col_groups
        ebase = shard * ew

        for i in range(ncols):
            coff = pl.multiple_of((cg * ncols + i) * n, 8)
            pltpu.sync_copy(tflat_hbm.at[pl.ds(coff, n)],
                            tvs[i].at[pl.ds(0, n)])
            pltpu.sync_copy(zeros_hbm, avs[i].at[pl.ds(0, n)])

        def start(k, sv, dv, sem_s, sem_d):
            off = pl.multiple_of(ebase + k * _CE, 8)
            pltpu.async_copy(src_hbm.at[pl.ds(off, _CE)], sv, sem_s)
            pltpu.async_copy(dst_hbm.at[pl.ds(off, _CE)], dv, sem_d)

        def wait(sv, dv, sem_s, sem_d):
            pltpu.make_async_copy(src_hbm.at[pl.ds(0, _CE)], sv, sem_s).wait()
            pltpu.make_async_copy(dst_hbm.at[pl.ds(0, _CE)], dv, sem_d).wait()

        def process(sv, dv):
            @plsc.parallel_loop(0, groups, unroll=8)
            def _gbody(g):
                goff = pl.multiple_of(g * 16, 16)
                si = sv[pl.ds(goff, 16)]
                di = dv[pl.ds(goff, 16)]
                for i in range(ncols):
                    vals = plsc.load_gather(tvs[i], [si])
                    plsc.addupdate_scatter(avs[i], [di], vals)

        if nchunks == 1:
            start(0, sv0, dv0, sems[0], sems[1])
            wait(sv0, dv0, sems[0], sems[1])
            process(sv0, dv0)
        else:
            start(0, sv0, dv0, sems[0], sems[1])

            def chunk2_body(k2, carry):
                wait(sv0, dv0, sems[0], sems[1])
                start(2 * k2 + 1, sv1, dv1, sems[2], sems[3])
                process(sv0, dv0)
                wait(sv1, dv1, sems[2], sems[3])

                @pl.when(2 * k2 + 2 < nchunks)
                def _start_next():
                    start(2 * k2 + 2, sv0, dv0, sems[0], sems[1])

                process(sv1, dv1)
                return carry

            lax.fori_loop(0, nchunks // 2, chunk2_body, 0)

        for i in range(ncols):
            ooff = pl.multiple_of((shard * d + cg * ncols + i) * n, 8)
            pltpu.sync_copy(avs[i].at[pl.ds(0, n)],
                            out_hbm.at[pl.ds(ooff, n)])

    return kern


def _edge_pass(n, d, e_pad, ncols=1):
    kern = _edge_pass_kernel(n, d, e_pad, ncols)
    wpc = _NWORKERS // (d // ncols)

    def run(table_t_flat, zeros, src_flat, dst_flat):
        out = kern(table_t_flat, zeros, src_flat, dst_flat)
        return out.reshape(wpc, d, n)

    return run


def _leaky(v):
    return jnp.where(v >= 0, v, 0.01 * v)


def _dgT(a, b):
    return lax.dot_general(a, b, (((0,), (0,)), ((), ())),
                           preferred_element_type=jnp.float32)


def _mm_k(x_ref, w_ref, o_ref):
    o_ref[...] = lax.dot_general(w_ref[...], x_ref[...],
                                 (((0,), (1,)), ((), ())),
                                 preferred_element_type=jnp.float32)


def _bn1_k(aggp_ref, b_ref, g_ref, be_ref, rb_ref, ow_ref,
           h_ref, ho_ref):
    v = jnp.sum(aggp_ref[...], axis=0) + b_ref[...][:, None]
    m = jnp.mean(v, axis=1, keepdims=True)
    var = jnp.mean(v * v, axis=1, keepdims=True) - m * m
    h = _leaky((v - m) * (g_ref[...][:, None] * lax.rsqrt(var + 1e-5))
               + be_ref[...][:, None])
    h_ref[...] = h
    ho_ref[...] = _dgT(ow_ref[...], h) + rb_ref[...][:, None]


def _mono_key(score):
    u = lax.bitcast_convert_type(score, jnp.uint32)
    return u ^ jnp.where(u >> 31 != 0,
                         jnp.uint32(0xFFFFFFFF), jnp.uint32(0x80000000))


def _topk_mask(score, valid, k):
    n = score.shape[1]
    key = _mono_key(score)

    def vbit(b, t):
        tt = t | (jnp.uint32(1) << b)
        cnt = jnp.sum(valid & (key >= tt))
        return jnp.where(cnt >= k, tt, t)

    t = lax.fori_loop(0, 32, lambda i, t: vbit(jnp.uint32(31 - i), t),
                      jnp.uint32(0))
    cnt_gt = jnp.sum(valid & (key > t))
    need = k - cnt_gt
    idx = lax.broadcasted_iota(jnp.int32, (1, n), 1)
    ties = valid & (key == t)

    def ibit(b, m):
        mm = m | (jnp.int32(1) << b)
        cnt = jnp.sum(ties & (idx < mm))
        return jnp.where(cnt < need, mm, m)

    m = lax.fori_loop(0, 15, lambda i, m: ibit(jnp.int32(14 - i), m),
                      jnp.int32(0))
    return valid & ((key > t) | (ties & (idx <= m)))


def _topk1_k(saggp_ref, rw_ref, ho_ref, h_ref, w2_ref, h2lin_ref, sel_ref,
             *, k):
    score = _dgT(rw_ref[...], jnp.sum(saggp_ref[...], axis=0)) + ho_ref[...]
    sel = _topk_mask(score, jnp.full(score.shape, True), k)
    xn = jnp.where(sel, h_ref[...] * jnp.tanh(score), 0.0)
    h2lin_ref[...] = _dgT(w2_ref[...], xn)
    sel_ref[...] = sel.astype(jnp.float32)


def _bn2_k(aggp_ref, sel_ref, b_ref, g_ref, be_ref, rb_ref, ow_ref,
           h2_ref, ho_ref, *, k):
    selv = sel_ref[...]
    v = jnp.sum(aggp_ref[...], axis=0) + b_ref[...][:, None]
    m = jnp.sum(v * selv, axis=1, keepdims=True) / k
    var = jnp.sum(v * v * selv, axis=1, keepdims=True) / k - m * m
    h2 = selv * _leaky((v - m) * (g_ref[...][:, None] * lax.rsqrt(var + 1e-5))
                       + be_ref[...][:, None])
    h2_ref[...] = h2
    ho_ref[...] = _dgT(ow_ref[...], h2) + rb_ref[...][:, None]


def _final_k(saggp_ref, rw_ref, ho_ref, sel_ref, h2_ref,
             f1w_ref, f1b_ref, f2w_ref, f2b_ref, f3w_ref, f3b_ref,
             o_ref, *, k):
    score = _dgT(rw_ref[...], jnp.sum(saggp_ref[...], axis=0)) + ho_ref[...]
    sel2 = _topk_mask(score, sel_ref[...] > 0, k)
    w = jnp.where(sel2, jnp.tanh(score), 0.0)
    ap = jnp.sum(h2_ref[...] * w, axis=1, keepdims=True)
    o = _leaky(_dgT(f1w_ref[...], ap) + f1b_ref[...][:, None])
    o = _leaky(_dgT(f2w_ref[...], o) + f2b_ref[...][:, None])
    o = _leaky(_dgT(f3w_ref[...], o) + f3b_ref[...][:, None])
    o_ref[...] = o


def _call(body, out_shapes):
    return pl.pallas_call(
        body, out_shape=[jax.ShapeDtypeStruct(s, jnp.float32)
                         for s in out_shapes])


def kernel(x, edge_index, edge_weigth, batch,
           W1, b1, g1, be1, p1rw, p1rb, p1ow,
           W2, b2, g2, be2, p2rw, p2rb, p2ow,
           f1w, f1b, f2w, f2b, f3w, f3b):
    n, _ = x.shape
    e = edge_index.shape[1]
    k1 = -(-n // 2)
    k2 = -(-k1 // 2)
    d1 = W1.shape[1]
    d2 = W2.shape[1]

    e_pad = -(-e // (_NWORKERS * _CE)) * (_NWORKERS * _CE)
    pad_idx = n + (jnp.arange(e_pad - e, dtype=jnp.int32) % _NSUB)
    src = jnp.concatenate([edge_index[0], pad_idx])
    dst = jnp.concatenate([edge_index[1], pad_idx])

    ep16 = _edge_pass(n, d1, e_pad, ncols=2)
    ep32 = _edge_pass(n, d2, e_pad, ncols=4)
    zeros = jnp.zeros((n,), jnp.float32)

    (hlinT,) = _call(_mm_k, [(d1, n)])(x, W1)
    agg1p = ep16(hlinT.reshape(-1), zeros, src, dst)
    hT, ho1 = _call(_bn1_k, [(d1, n), (1, n)])(
        agg1p, b1, g1, be1, p1rb, p1ow)

    sagg1p = ep16(hT.reshape(-1), zeros, src, dst)
    h2linT, sel1 = _call(functools.partial(_topk1_k, k=k1),
                         [(d2, n), (1, n)])(sagg1p, p1rw, ho1, hT, W2)

    agg2p = ep32(h2linT.reshape(-1), zeros, src, dst)
    h2T, ho2 = _call(functools.partial(_bn2_k, k=k1),
                     [(d2, n), (1, n)])(
        agg2p, sel1, b2, g2, be2, p2rb, p2ow)

    sagg2p = ep32(h2T.reshape(-1), zeros, src, dst)
    (o,) = _call(functools.partial(_final_k, k=k2), [(2, 1)])(
        sagg2p, p2rw, ho2, sel1, h2T, f1w, f1b, f2w, f2b, f3w, f3b)
    return o.T
